# baseline (speedup 1.0000x reference)
import functools

import jax
import jax.numpy as jnp
from jax import lax
from jax.experimental import pallas as pl
from jax.experimental.pallas import tpu as pltpu
from jax.experimental.pallas import tpu_sc as plsc

N = 10000
NP = 10240
E = 320000
EP = 327680
D = 128
CK = 64
NTILE = 16
NPT = NP // NTILE
ECH_ALL = EP // NTILE
ECH_HALF = EP // (2 * NTILE)
NCH_ALL = ECH_ALL // CK
NCH_HALF = ECH_HALF // CK
DEPTH = 8
RB = 512
GRID = NP // RB


def _sc_mesh():
    return plsc.VectorSubcoreMesh(core_axis_name="c", subcore_axis_name="s")


def _drain(sem, dummy_hbm, dst_ref):
    pltpu.make_async_copy(dummy_hbm, dst_ref, sem).wait()


def _prep_body(dst1_hbm, dst2_hbm, src2_hbm, dis_hbm, up_hbm,
               zb, degb, onesb, dall, uda, usa, val0, val1,
               semA, semB, sg0, sg1, ss0, ss1,
               deg_sp, dis_sp, u_sp):
    c = lax.axis_index("c")
    s = lax.axis_index("s")
    row0 = s * NPT

    pltpu.async_copy(dst2_hbm.at[pl.ds(s * NCH_ALL, NCH_ALL)], dall, semA)
    wid = c * NTILE + s
    pltpu.async_copy(dst1_hbm.at[pl.ds(wid * ECH_HALF, ECH_HALF)], uda, semB)
    pltpu.async_copy(src2_hbm.at[pl.ds(wid * NCH_HALF, NCH_HALF)], usa, semB)

    for i in range(NPT // 16):
        zb[pl.ds(i * 16, 16)] = jnp.zeros((16,), jnp.float32)
    for i in range(CK // 16):
        onesb[pl.ds(i * 16, 16)] = jnp.ones((16,), jnp.float32)
    pltpu.sync_copy(zb, deg_sp.at[pl.ds(row0, NPT)])
    pltpu.sync_copy(zb, u_sp.at[pl.ds(row0, NPT)])
    _drain(semA, dst2_hbm.at[pl.ds(0, NCH_ALL)], dall)
    _drain(semB, dst1_hbm.at[pl.ds(0, ECH_HALF)], uda)
    _drain(semB, src2_hbm.at[pl.ds(0, NCH_HALF)], usa)
    plsc.subcore_barrier()

    def deg_step(k, carry):
        pltpu.async_copy(onesb, deg_sp.at[dall.at[k]], semA, add=True)

        @pl.when(k >= DEPTH)
        def _():
            _drain(semA, dis_hbm.at[pl.ds(0, CK)], onesb)

        return carry

    lax.fori_loop(0, NCH_ALL, deg_step, 0)
    for _ in range(DEPTH):
        _drain(semA, dis_hbm.at[pl.ds(0, CK)], onesb)
    plsc.subcore_barrier()

    pltpu.sync_copy(deg_sp.at[pl.ds(row0, NPT)], degb)
    for i in range(NPT // 16):
        d = degb[pl.ds(i * 16, 16)] + 1.0
        ii = lax.bitcast_convert_type(d, jnp.int32)
        ii = jnp.int32(0x5F3759DF) - lax.shift_right_logical(ii, 1)
        y = lax.bitcast_convert_type(ii, jnp.float32)
        half = d * 0.5
        y = y * (1.5 - half * y * y)
        y = y * (1.5 - half * y * y)
        y = y * (1.5 - half * y * y)
        gidx = row0 + i * 16 + lax.iota(jnp.int32, 16)
        y = jnp.where(gidx < N, y, 0.0)
        degb[pl.ds(i * 16, 16)] = y
    pltpu.sync_copy(degb, dis_sp.at[pl.ds(row0, NPT)])

    @pl.when(c == 0)
    def _():
        pltpu.sync_copy(degb, dis_hbm.at[pl.ds(row0, NPT)])

    plsc.subcore_barrier()

    vals = (val0, val1)
    sg = (sg0, sg1)
    ss = (ss0, ss1)
    pltpu.async_copy(dis_sp.at[uda.at[pl.ds(0, CK)]], val0, sg0)

    def u_pair(j, carry):
        for b in (0, 1):
            kk = 2 * j + b
            nb = 1 - b
            _drain(sg[b], dis_hbm.at[pl.ds(0, CK)], vals[b])
            pltpu.async_copy(vals[b], u_sp.at[usa.at[kk]], ss[b], add=True)

            @pl.when(kk >= 1)
            def _():
                _drain(ss[nb], dis_hbm.at[pl.ds(0, CK)], vals[nb])

            @pl.when(kk + 1 < NCH_HALF)
            def _():
                pltpu.async_copy(dis_sp.at[uda.at[pl.ds((kk + 1) * CK, CK)]],
                                 vals[nb], sg[nb])
        return carry

    lax.fori_loop(0, NCH_HALF // 2, u_pair, 0)
    _drain(ss[1], dis_hbm.at[pl.ds(0, CK)], val1)
    plsc.subcore_barrier()
    pltpu.sync_copy(u_sp.at[pl.ds(row0, NPT)],
                    up_hbm.at[c, 0, pl.ds(row0, NPT)])


_prep = functools.partial(
    pl.kernel,
    out_type=(jax.ShapeDtypeStruct((NP,), jnp.float32),
              jax.ShapeDtypeStruct((2, 1, NP), jnp.float32)),
    mesh=_sc_mesh(),
    scratch_types=[
        pltpu.VMEM((NPT,), jnp.float32),
        pltpu.VMEM((NPT,), jnp.float32),
        pltpu.VMEM((CK,), jnp.float32),
        pltpu.VMEM((NCH_ALL, CK), jnp.int32),
        pltpu.VMEM((ECH_HALF,), jnp.int32),
        pltpu.VMEM((NCH_HALF, CK), jnp.int32),
        pltpu.VMEM((CK,), jnp.float32),
        pltpu.VMEM((CK,), jnp.float32),
        pltpu.SemaphoreType.DMA,
        pltpu.SemaphoreType.DMA,
        pltpu.SemaphoreType.DMA,
        pltpu.SemaphoreType.DMA,
        pltpu.SemaphoreType.DMA,
        pltpu.SemaphoreType.DMA,
        pltpu.VMEM_SHARED((NP,), jnp.float32),
        pltpu.VMEM_SHARED((NP,), jnp.float32),
        pltpu.VMEM_SHARED((NP,), jnp.float32),
    ],
)(_prep_body)


def _row_body(t_hbm, src1_hbm, dst2_hbm, out_hbm,
              srcall, dstall, rows0, rows1,
              semA, semB, sg0, sg1, ss0, ss1, acc_sp):
    c = lax.axis_index("c")
    s = lax.axis_index("s")
    row0 = s * NPT
    wid = c * NTILE + s

    pltpu.async_copy(src1_hbm.at[pl.ds(wid * ECH_HALF, ECH_HALF)],
                     srcall, semA)
    pltpu.async_copy(dst2_hbm.at[pl.ds(wid * NCH_HALF, NCH_HALF)],
                     dstall, semB)
    pltpu.sync_copy(t_hbm.at[pl.ds(row0, NPT)], acc_sp.at[pl.ds(row0, NPT)])
    _drain(semA, src1_hbm.at[pl.ds(0, ECH_HALF)], srcall)
    _drain(semB, dst2_hbm.at[pl.ds(0, NCH_HALF)], dstall)
    plsc.subcore_barrier()

    rows = (rows0, rows1)
    sg = (sg0, sg1)
    ss = (ss0, ss1)
    pltpu.async_copy(t_hbm.at[srcall.at[pl.ds(0, CK)]], rows0, sg0)

    def pair(j, carry):
        for b in (0, 1):
            kk = 2 * j + b
            nb = 1 - b
            _drain(sg[b], t_hbm.at[pl.ds(0, CK)], rows[b])
            pltpu.async_copy(rows[b], acc_sp.at[dstall.at[kk]], ss[b],
                             add=True)

            @pl.when(kk >= 1)
            def _():
                _drain(ss[nb], t_hbm.at[pl.ds(0, CK)], rows[nb])

            @pl.when(kk + 1 < NCH_HALF)
            def _():
                pltpu.async_copy(t_hbm.at[srcall.at[pl.ds((kk + 1) * CK, CK)]],
                                 rows[nb], sg[nb])
        return carry

    lax.fori_loop(0, NCH_HALF // 2, pair, 0)
    _drain(ss[1], t_hbm.at[pl.ds(0, CK)], rows1)
    plsc.subcore_barrier()
    pltpu.sync_copy(acc_sp.at[pl.ds(row0, NPT)],
                    out_hbm.at[c, pl.ds(row0, NPT)])


_rowpass = functools.partial(
    pl.kernel,
    out_type=jax.ShapeDtypeStruct((2, NP, D), jnp.float32),
    mesh=_sc_mesh(),
    scratch_types=[
        pltpu.VMEM((ECH_HALF,), jnp.int32),
        pltpu.VMEM((NCH_HALF, CK), jnp.int32),
        pltpu.VMEM((CK, D), jnp.float32),
        pltpu.VMEM((CK, D), jnp.float32),
        pltpu.SemaphoreType.DMA,
        pltpu.SemaphoreType.DMA,
        pltpu.SemaphoreType.DMA,
        pltpu.SemaphoreType.DMA,
        pltpu.SemaphoreType.DMA,
        pltpu.SemaphoreType.DMA,
        pltpu.VMEM_SHARED((NP, D), jnp.float32),
    ],
)(_row_body)


def _mm1_body(x_ref, w_ref, dis_ref, o_ref):
    o_ref[...] = (jnp.dot(x_ref[...], w_ref[...],
                          preferred_element_type=jnp.float32) * dis_ref[...])


def _mm1(x_p, W1, dis_col):
    return pl.pallas_call(
        _mm1_body,
        grid=(GRID,),
        in_specs=[
            pl.BlockSpec((RB, D), lambda i: (i, 0)),
            pl.BlockSpec((D, D), lambda i: (0, 0)),
            pl.BlockSpec((RB, 1), lambda i: (i, 0)),
        ],
        out_specs=pl.BlockSpec((RB, D), lambda i: (i, 0)),
        out_shape=jax.ShapeDtypeStruct((NP, D), jnp.float32),
    )(x_p, W1, dis_col)


def _mid_body(a0_ref, a1_ref, t_ref, dis_ref, u0_ref, u1_ref, b_ref, w_ref,
              t2_ref, v_ref):
    dis = dis_ref[...]
    q = a0_ref[...] + a1_ref[...] - t_ref[...]
    h = jnp.maximum(dis * q + b_ref[...], 0.0)
    t2_ref[...] = jnp.dot(h, w_ref[...],
                          preferred_element_type=jnp.float32) * dis
    v_ref[...] = dis * (u0_ref[...] + u1_ref[...] + dis)


def _mid(acc1, t1, dis_col, u0, u1, b1, W2):
    return pl.pallas_call(
        _mid_body,
        grid=(GRID,),
        in_specs=[
            pl.BlockSpec((RB, D), lambda i: (i, 0)),
            pl.BlockSpec((RB, D), lambda i: (i, 0)),
            pl.BlockSpec((RB, D), lambda i: (i, 0)),
            pl.BlockSpec((RB, 1), lambda i: (i, 0)),
            pl.BlockSpec((RB, 1), lambda i: (i, 0)),
            pl.BlockSpec((RB, 1), lambda i: (i, 0)),
            pl.BlockSpec((1, D), lambda i: (0, 0)),
            pl.BlockSpec((D, D), lambda i: (0, 0)),
        ],
        out_specs=[
            pl.BlockSpec((RB, D), lambda i: (i, 0)),
            pl.BlockSpec((RB, 1), lambda i: (i, 0)),
        ],
        out_shape=[
            jax.ShapeDtypeStruct((NP, D), jnp.float32),
            jax.ShapeDtypeStruct((NP, 1), jnp.float32),
        ],
    )(acc1[0], acc1[1], t1, dis_col, u0, u1, b1, W2)


def _head_body(a0_ref, a1_ref, t_ref, dis_ref, v_ref, b2_ref, w3_ref, b3_ref,
               wc1_ref, bc1_ref, wc2_ref, bc2_ref, acc_ref, o_ref):
    i = pl.program_id(0)
    dis = dis_ref[...]
    q = a0_ref[...] + a1_ref[...] - t_ref[...]
    h = jnp.maximum(dis * q + b2_ref[...], 0.0)
    contrib = jnp.sum(v_ref[...] * h, axis=0, keepdims=True)

    @pl.when(i == 0)
    def _():
        acc_ref[...] = contrib

    @pl.when(i > 0)
    def _():
        acc_ref[...] = acc_ref[...] + contrib

    @pl.when(i == GRID - 1)
    def _():
        g = (jnp.dot(acc_ref[...], w3_ref[...],
                     preferred_element_type=jnp.float32) * (1.0 / N)
             + b3_ref[...])
        z = jnp.maximum(jnp.dot(g, wc1_ref[...],
                                preferred_element_type=jnp.float32)
                        + bc1_ref[...], 0.0)
        o = jnp.dot(z, wc2_ref[...],
                    preferred_element_type=jnp.float32) + bc2_ref[...]
        o_ref[...] = 1.0 / (1.0 + jnp.exp(-o))


def _head(acc2, t2, dis_col, v_col, b2, W3, b3, Wc1, bc1, Wc2p, bc2p):
    return pl.pallas_call(
        _head_body,
        grid=(GRID,),
        in_specs=[
            pl.BlockSpec((RB, D), lambda i: (i, 0)),
            pl.BlockSpec((RB, D), lambda i: (i, 0)),
            pl.BlockSpec((RB, D), lambda i: (i, 0)),
            pl.BlockSpec((RB, 1), lambda i: (i, 0)),
            pl.BlockSpec((RB, 1), lambda i: (i, 0)),
            pl.BlockSpec((1, D), lambda i: (0, 0)),
            pl.BlockSpec((D, D), lambda i: (0, 0)),
            pl.BlockSpec((1, D), lambda i: (0, 0)),
            pl.BlockSpec((D, 64), lambda i: (0, 0)),
            pl.BlockSpec((1, 64), lambda i: (0, 0)),
            pl.BlockSpec((64, D), lambda i: (0, 0)),
            pl.BlockSpec((1, D), lambda i: (0, 0)),
        ],
        out_specs=[
            pl.BlockSpec((1, D), lambda i: (0, 0)),
            pl.BlockSpec((1, D), lambda i: (0, 0)),
        ],
        out_shape=[
            jax.ShapeDtypeStruct((1, D), jnp.float32),
            jax.ShapeDtypeStruct((1, D), jnp.float32),
        ],
    )(acc2[0], acc2[1], t2, dis_col, v_col, b2, W3, b3, Wc1, bc1, Wc2p, bc2p)


def kernel(x, edge_index, W1, b1, W2, b2, W3, b3, Wc1, bc1, Wc2, bc2):
    src = edge_index[0]
    dst = edge_index[1]
    padv = (N + jnp.arange(EP - E, dtype=jnp.int32) % (NP - N))
    src_p = jnp.concatenate([src, padv])
    dst_p = jnp.concatenate([dst, padv])
    src2 = src_p.reshape(EP // CK, CK)
    dst2 = dst_p.reshape(EP // CK, CK)
    x_p = jnp.pad(x, ((0, NP - N), (0, 0)))

    dis, up = _prep(dst_p, dst2, src2)
    dis_col = dis.reshape(NP, 1)
    u0 = up[0, 0].reshape(NP, 1)
    u1 = up[1, 0].reshape(NP, 1)

    t1 = _mm1(x_p, W1, dis_col)
    acc1 = _rowpass(t1, src_p, dst2)
    t2, v_col = _mid(acc1, t1, dis_col, u0, u1, b1.reshape(1, D), W2)
    acc2 = _rowpass(t2, src_p, dst2)

    Wc2p = jnp.pad(Wc2, ((0, 0), (0, D - 6)))
    bc2p = jnp.pad(bc2, (0, D - 6)).reshape(1, D)
    _, outp = _head(acc2, t2, dis_col, v_col, b2.reshape(1, D), W3,
                    b3.reshape(1, D), Wc1, bc1.reshape(1, 64), Wc2p, bc2p)
    return outp[:, :6]


# speedup vs baseline: 1.2251x; 1.2251x over previous
import functools

import jax
import jax.numpy as jnp
from jax import lax
from jax.experimental import pallas as pl
from jax.experimental.pallas import tpu as pltpu
from jax.experimental.pallas import tpu_sc as plsc

N = 10000
NP = 10240
E = 320000
EP = 327680
D = 128
CK = 64
NTILE = 16
NPT = NP // NTILE
ECH_ALL = EP // NTILE
ECH_HALF = EP // (2 * NTILE)
NCH_ALL = ECH_ALL // CK
NCH_HALF = ECH_HALF // CK
DEPTH = 8
RB = 512
GRID = NP // RB


def _sc_mesh():
    return plsc.VectorSubcoreMesh(core_axis_name="c", subcore_axis_name="s")


def _drain(sem, dummy_hbm, dst_ref):
    pltpu.make_async_copy(dummy_hbm, dst_ref, sem).wait()


def _prep_body(dst1_hbm, dst2_hbm, src2_hbm, dis_hbm, up_hbm,
               zb, degb, onesb, dall, uda, usa, val0, val1,
               semA, semB, sg0, sg1, ss0, ss1,
               deg_sp, dis_sp, u_sp):
    c = lax.axis_index("c")
    s = lax.axis_index("s")
    row0 = s * NPT

    pltpu.async_copy(dst2_hbm.at[pl.ds(s * NCH_ALL, NCH_ALL)], dall, semA)
    wid = c * NTILE + s
    pltpu.async_copy(dst1_hbm.at[pl.ds(wid * ECH_HALF, ECH_HALF)], uda, semB)
    pltpu.async_copy(src2_hbm.at[pl.ds(wid * NCH_HALF, NCH_HALF)], usa, semB)

    for i in range(NPT // 16):
        zb[pl.ds(i * 16, 16)] = jnp.zeros((16,), jnp.float32)
    for i in range(CK // 16):
        onesb[pl.ds(i * 16, 16)] = jnp.ones((16,), jnp.float32)
    pltpu.sync_copy(zb, deg_sp.at[pl.ds(row0, NPT)])
    pltpu.sync_copy(zb, u_sp.at[pl.ds(row0, NPT)])
    _drain(semA, dst2_hbm.at[pl.ds(0, NCH_ALL)], dall)
    _drain(semB, dst1_hbm.at[pl.ds(0, ECH_HALF)], uda)
    _drain(semB, src2_hbm.at[pl.ds(0, NCH_HALF)], usa)
    plsc.subcore_barrier()

    def deg_step(k, carry):
        pltpu.async_copy(onesb, deg_sp.at[dall.at[k]], semA, add=True)

        @pl.when(k >= DEPTH)
        def _():
            _drain(semA, dis_hbm.at[pl.ds(0, CK)], onesb)

        return carry

    lax.fori_loop(0, NCH_ALL, deg_step, 0)
    for _ in range(DEPTH):
        _drain(semA, dis_hbm.at[pl.ds(0, CK)], onesb)
    plsc.subcore_barrier()

    pltpu.sync_copy(deg_sp.at[pl.ds(row0, NPT)], degb)
    for i in range(NPT // 16):
        d = degb[pl.ds(i * 16, 16)] + 1.0
        ii = lax.bitcast_convert_type(d, jnp.int32)
        ii = jnp.int32(0x5F3759DF) - lax.shift_right_logical(ii, 1)
        y = lax.bitcast_convert_type(ii, jnp.float32)
        half = d * 0.5
        y = y * (1.5 - half * y * y)
        y = y * (1.5 - half * y * y)
        y = y * (1.5 - half * y * y)
        gidx = row0 + i * 16 + lax.iota(jnp.int32, 16)
        y = jnp.where(gidx < N, y, 0.0)
        degb[pl.ds(i * 16, 16)] = y
    pltpu.sync_copy(degb, dis_sp.at[pl.ds(row0, NPT)])

    @pl.when(c == 0)
    def _():
        pltpu.sync_copy(degb, dis_hbm.at[pl.ds(row0, NPT)])

    plsc.subcore_barrier()

    vals = (val0, val1)
    sg = (sg0, sg1)
    ss = (ss0, ss1)
    pltpu.async_copy(dis_sp.at[uda.at[pl.ds(0, CK)]], val0, sg0)

    def u_pair(j, carry):
        for b in (0, 1):
            kk = 2 * j + b
            nb = 1 - b
            _drain(sg[b], dis_hbm.at[pl.ds(0, CK)], vals[b])
            pltpu.async_copy(vals[b], u_sp.at[usa.at[kk]], ss[b], add=True)

            @pl.when(kk >= 1)
            def _():
                _drain(ss[nb], dis_hbm.at[pl.ds(0, CK)], vals[nb])

            @pl.when(kk + 1 < NCH_HALF)
            def _():
                pltpu.async_copy(dis_sp.at[uda.at[pl.ds((kk + 1) * CK, CK)]],
                                 vals[nb], sg[nb])
        return carry

    lax.fori_loop(0, NCH_HALF // 2, u_pair, 0)
    _drain(ss[1], dis_hbm.at[pl.ds(0, CK)], val1)
    plsc.subcore_barrier()
    pltpu.sync_copy(u_sp.at[pl.ds(row0, NPT)],
                    up_hbm.at[c, 0, pl.ds(row0, NPT)])


_prep = functools.partial(
    pl.kernel,
    out_type=(jax.ShapeDtypeStruct((NP,), jnp.float32),
              jax.ShapeDtypeStruct((2, 1, NP), jnp.float32)),
    mesh=_sc_mesh(),
    scratch_types=[
        pltpu.VMEM((NPT,), jnp.float32),
        pltpu.VMEM((NPT,), jnp.float32),
        pltpu.VMEM((CK,), jnp.float32),
        pltpu.VMEM((NCH_ALL, CK), jnp.int32),
        pltpu.VMEM((ECH_HALF,), jnp.int32),
        pltpu.VMEM((NCH_HALF, CK), jnp.int32),
        pltpu.VMEM((CK,), jnp.float32),
        pltpu.VMEM((CK,), jnp.float32),
        pltpu.SemaphoreType.DMA,
        pltpu.SemaphoreType.DMA,
        pltpu.SemaphoreType.DMA,
        pltpu.SemaphoreType.DMA,
        pltpu.SemaphoreType.DMA,
        pltpu.SemaphoreType.DMA,
        pltpu.VMEM_SHARED((NP,), jnp.float32),
        pltpu.VMEM_SHARED((NP,), jnp.float32),
        pltpu.VMEM_SHARED((NP,), jnp.float32),
    ],
)(_prep_body)


CKS = 128
NCHS = ECH_HALF // CKS
NCHS_H = NCHS // 2


def _row_body(t_hbm, src1_hbm, dst2_hbm, out_hbm,
              srcall, dstall, rows0, rows1,
              semA, semB, sg0, sg1, ss0, ss1, acc_sp):
    c = lax.axis_index("c")
    s = lax.axis_index("s")
    row0 = s * NPT
    wid = c * NTILE + s
    e0 = wid * ECH_HALF
    half_edges = ECH_HALF // 2

    pltpu.async_copy(src1_hbm.at[pl.ds(e0, half_edges)], srcall, semA)
    pltpu.async_copy(dst2_hbm.at[pl.ds(wid * NCHS, NCHS)], dstall, semB)
    pltpu.sync_copy(t_hbm.at[pl.ds(row0, NPT)], acc_sp.at[pl.ds(row0, NPT)])
    _drain(semA, src1_hbm.at[pl.ds(0, half_edges)], srcall)
    _drain(semB, dst2_hbm.at[pl.ds(0, NCHS)], dstall)
    plsc.subcore_barrier()

    rows = (rows0, rows1)
    sg = (sg0, sg1)
    ss = (ss0, ss1)

    for h in (0, 1):
        if h == 1:
            pltpu.async_copy(src1_hbm.at[pl.ds(e0 + half_edges, half_edges)],
                             srcall, semA)
            _drain(semA, src1_hbm.at[pl.ds(0, half_edges)], srcall)
        base = h * NCHS_H
        pltpu.async_copy(t_hbm.at[srcall.at[pl.ds(0, CKS)]], rows0, sg0)

        def pair(j, carry):
            for b in (0, 1):
                lk = 2 * j + b
                nb = 1 - b
                _drain(sg[b], t_hbm.at[pl.ds(0, CKS)], rows[b])
                pltpu.async_copy(rows[b], acc_sp.at[dstall.at[base + lk]],
                                 ss[b], add=True)

                @pl.when(lk >= 1)
                def _():
                    _drain(ss[nb], t_hbm.at[pl.ds(0, CKS)], rows[nb])

                @pl.when(lk + 1 < NCHS_H)
                def _():
                    pltpu.async_copy(
                        t_hbm.at[srcall.at[pl.ds((lk + 1) * CKS, CKS)]],
                        rows[nb], sg[nb])
            return carry

        lax.fori_loop(0, NCHS_H // 2, pair, 0)
        _drain(ss[1], t_hbm.at[pl.ds(0, CKS)], rows1)

    plsc.subcore_barrier()
    pltpu.sync_copy(acc_sp.at[pl.ds(row0, NPT)],
                    out_hbm.at[c, pl.ds(row0, NPT)])


_rowpass = functools.partial(
    pl.kernel,
    out_type=jax.ShapeDtypeStruct((2, NP, D), jnp.float32),
    mesh=_sc_mesh(),
    scratch_types=[
        pltpu.VMEM((ECH_HALF // 2,), jnp.int32),
        pltpu.VMEM((NCHS, CKS), jnp.int32),
        pltpu.VMEM((CKS, D), jnp.float32),
        pltpu.VMEM((CKS, D), jnp.float32),
        pltpu.SemaphoreType.DMA,
        pltpu.SemaphoreType.DMA,
        pltpu.SemaphoreType.DMA,
        pltpu.SemaphoreType.DMA,
        pltpu.SemaphoreType.DMA,
        pltpu.SemaphoreType.DMA,
        pltpu.VMEM_SHARED((NP, D), jnp.float32),
    ],
)(_row_body)


def _mm1_body(x_ref, w_ref, dis_ref, o_ref):
    o_ref[...] = (jnp.dot(x_ref[...], w_ref[...],
                          preferred_element_type=jnp.float32) * dis_ref[...])


def _mm1(x_p, W1, dis_col):
    return pl.pallas_call(
        _mm1_body,
        grid=(GRID,),
        in_specs=[
            pl.BlockSpec((RB, D), lambda i: (i, 0)),
            pl.BlockSpec((D, D), lambda i: (0, 0)),
            pl.BlockSpec((RB, 1), lambda i: (i, 0)),
        ],
        out_specs=pl.BlockSpec((RB, D), lambda i: (i, 0)),
        out_shape=jax.ShapeDtypeStruct((NP, D), jnp.float32),
    )(x_p, W1, dis_col)


def _mid_body(a0_ref, a1_ref, t_ref, dis_ref, u0_ref, u1_ref, b_ref, w_ref,
              t2_ref, v_ref):
    dis = dis_ref[...]
    q = a0_ref[...] + a1_ref[...] - t_ref[...]
    h = jnp.maximum(dis * q + b_ref[...], 0.0)
    t2_ref[...] = jnp.dot(h, w_ref[...],
                          preferred_element_type=jnp.float32) * dis
    v_ref[...] = dis * (u0_ref[...] + u1_ref[...] + dis)


def _mid(acc1, t1, dis_col, u0, u1, b1, W2):
    return pl.pallas_call(
        _mid_body,
        grid=(GRID,),
        in_specs=[
            pl.BlockSpec((RB, D), lambda i: (i, 0)),
            pl.BlockSpec((RB, D), lambda i: (i, 0)),
            pl.BlockSpec((RB, D), lambda i: (i, 0)),
            pl.BlockSpec((RB, 1), lambda i: (i, 0)),
            pl.BlockSpec((RB, 1), lambda i: (i, 0)),
            pl.BlockSpec((RB, 1), lambda i: (i, 0)),
            pl.BlockSpec((1, D), lambda i: (0, 0)),
            pl.BlockSpec((D, D), lambda i: (0, 0)),
        ],
        out_specs=[
            pl.BlockSpec((RB, D), lambda i: (i, 0)),
            pl.BlockSpec((RB, 1), lambda i: (i, 0)),
        ],
        out_shape=[
            jax.ShapeDtypeStruct((NP, D), jnp.float32),
            jax.ShapeDtypeStruct((NP, 1), jnp.float32),
        ],
    )(acc1[0], acc1[1], t1, dis_col, u0, u1, b1, W2)


def _head_body(a0_ref, a1_ref, t_ref, dis_ref, v_ref, b2_ref, w3_ref, b3_ref,
               wc1_ref, bc1_ref, wc2_ref, bc2_ref, acc_ref, o_ref):
    i = pl.program_id(0)
    dis = dis_ref[...]
    q = a0_ref[...] + a1_ref[...] - t_ref[...]
    h = jnp.maximum(dis * q + b2_ref[...], 0.0)
    contrib = jnp.sum(v_ref[...] * h, axis=0, keepdims=True)

    @pl.when(i == 0)
    def _():
        acc_ref[...] = contrib

    @pl.when(i > 0)
    def _():
        acc_ref[...] = acc_ref[...] + contrib

    @pl.when(i == GRID - 1)
    def _():
        g = (jnp.dot(acc_ref[...], w3_ref[...],
                     preferred_element_type=jnp.float32) * (1.0 / N)
             + b3_ref[...])
        z = jnp.maximum(jnp.dot(g, wc1_ref[...],
                                preferred_element_type=jnp.float32)
                        + bc1_ref[...], 0.0)
        o = jnp.dot(z, wc2_ref[...],
                    preferred_element_type=jnp.float32) + bc2_ref[...]
        o_ref[...] = 1.0 / (1.0 + jnp.exp(-o))


def _head(acc2, t2, dis_col, v_col, b2, W3, b3, Wc1, bc1, Wc2p, bc2p):
    return pl.pallas_call(
        _head_body,
        grid=(GRID,),
        in_specs=[
            pl.BlockSpec((RB, D), lambda i: (i, 0)),
            pl.BlockSpec((RB, D), lambda i: (i, 0)),
            pl.BlockSpec((RB, D), lambda i: (i, 0)),
            pl.BlockSpec((RB, 1), lambda i: (i, 0)),
            pl.BlockSpec((RB, 1), lambda i: (i, 0)),
            pl.BlockSpec((1, D), lambda i: (0, 0)),
            pl.BlockSpec((D, D), lambda i: (0, 0)),
            pl.BlockSpec((1, D), lambda i: (0, 0)),
            pl.BlockSpec((D, 64), lambda i: (0, 0)),
            pl.BlockSpec((1, 64), lambda i: (0, 0)),
            pl.BlockSpec((64, D), lambda i: (0, 0)),
            pl.BlockSpec((1, D), lambda i: (0, 0)),
        ],
        out_specs=[
            pl.BlockSpec((1, D), lambda i: (0, 0)),
            pl.BlockSpec((1, D), lambda i: (0, 0)),
        ],
        out_shape=[
            jax.ShapeDtypeStruct((1, D), jnp.float32),
            jax.ShapeDtypeStruct((1, D), jnp.float32),
        ],
    )(acc2[0], acc2[1], t2, dis_col, v_col, b2, W3, b3, Wc1, bc1, Wc2p, bc2p)


def kernel(x, edge_index, W1, b1, W2, b2, W3, b3, Wc1, bc1, Wc2, bc2):
    src = edge_index[0]
    dst = edge_index[1]
    padv = (N + jnp.arange(EP - E, dtype=jnp.int32) % (NP - N))
    src_p = jnp.concatenate([src, padv])
    dst_p = jnp.concatenate([dst, padv])
    src2 = src_p.reshape(EP // CK, CK)
    dst2 = dst_p.reshape(EP // CK, CK)
    dst2b = dst_p.reshape(EP // CKS, CKS)
    x_p = jnp.pad(x, ((0, NP - N), (0, 0)))

    dis, up = _prep(dst_p, dst2, src2)
    dis_col = dis.reshape(NP, 1)
    u0 = up[0, 0].reshape(NP, 1)
    u1 = up[1, 0].reshape(NP, 1)

    t1 = _mm1(x_p, W1, dis_col)
    acc1 = _rowpass(t1, src_p, dst2b)
    t2, v_col = _mid(acc1, t1, dis_col, u0, u1, b1.reshape(1, D), W2)
    acc2 = _rowpass(t2, src_p, dst2b)

    Wc2p = jnp.pad(Wc2, ((0, 0), (0, D - 6)))
    bc2p = jnp.pad(bc2, (0, D - 6)).reshape(1, D)
    _, outp = _head(acc2, t2, dis_col, v_col, b2.reshape(1, D), W3,
                    b3.reshape(1, D), Wc1, bc1.reshape(1, 64), Wc2p, bc2p)
    return outp[:, :6]


# speedup vs baseline: 1.2738x; 1.0398x over previous
import functools

import jax
import jax.numpy as jnp
from jax import lax
from jax.experimental import pallas as pl
from jax.experimental.pallas import tpu as pltpu
from jax.experimental.pallas import tpu_sc as plsc

N = 10000
NP = 10240
E = 320000
EP = 327680
D = 128
CK = 128
NTILE = 16
NPT = NP // NTILE
ECH_ALL = EP // NTILE
ECH_HALF = EP // (2 * NTILE)
NCH_ALL = ECH_ALL // CK
NCH_HALF = ECH_HALF // CK
DEPTH = 8
RB = 512
GRID = NP // RB


def _sc_mesh():
    return plsc.VectorSubcoreMesh(core_axis_name="c", subcore_axis_name="s")


def _drain(sem, dummy_hbm, dst_ref):
    pltpu.make_async_copy(dummy_hbm, dst_ref, sem).wait()


def _prep_body(dst2_hbm, src2_hbm, dis_hbm, up_hbm,
               zb, degb, onesb, dall, uda, usa, val0, val1,
               semA, semB, sg0, sg1, ss0, ss1,
               deg_sp, dis_sp, u_sp):
    c = lax.axis_index("c")
    s = lax.axis_index("s")
    row0 = s * NPT

    pltpu.async_copy(dst2_hbm.at[pl.ds(s * NCH_ALL, NCH_ALL)], dall, semA)
    wid = c * NTILE + s
    pltpu.async_copy(dst2_hbm.at[pl.ds(wid * NCH_HALF, NCH_HALF)], uda, semB)
    pltpu.async_copy(src2_hbm.at[pl.ds(wid * NCH_HALF, NCH_HALF)], usa, semB)

    for i in range(NPT // 16):
        zb[pl.ds(i * 16, 16)] = jnp.zeros((16,), jnp.float32)
    for i in range(CK // 16):
        onesb[pl.ds(i * 16, 16)] = jnp.ones((16,), jnp.float32)
    pltpu.sync_copy(zb, deg_sp.at[pl.ds(row0, NPT)])
    pltpu.sync_copy(zb, u_sp.at[pl.ds(row0, NPT)])
    _drain(semA, dst2_hbm.at[pl.ds(0, NCH_ALL)], dall)
    _drain(semB, dst2_hbm.at[pl.ds(0, NCH_HALF)], uda)
    _drain(semB, src2_hbm.at[pl.ds(0, NCH_HALF)], usa)
    plsc.subcore_barrier()

    def deg_step(k, carry):
        pltpu.async_copy(onesb, deg_sp.at[dall.at[k]], semA, add=True)

        @pl.when(k >= DEPTH)
        def _():
            _drain(semA, dis_hbm.at[pl.ds(0, CK)], onesb)

        return carry

    lax.fori_loop(0, NCH_ALL, deg_step, 0)
    for _ in range(DEPTH):
        _drain(semA, dis_hbm.at[pl.ds(0, CK)], onesb)
    plsc.subcore_barrier()

    pltpu.sync_copy(deg_sp.at[pl.ds(row0, NPT)], degb)
    for i in range(NPT // 16):
        d = degb[pl.ds(i * 16, 16)] + 1.0
        ii = lax.bitcast_convert_type(d, jnp.int32)
        ii = jnp.int32(0x5F3759DF) - lax.shift_right_logical(ii, 1)
        y = lax.bitcast_convert_type(ii, jnp.float32)
        half = d * 0.5
        y = y * (1.5 - half * y * y)
        y = y * (1.5 - half * y * y)
        y = y * (1.5 - half * y * y)
        gidx = row0 + i * 16 + lax.iota(jnp.int32, 16)
        y = jnp.where(gidx < N, y, 0.0)
        degb[pl.ds(i * 16, 16)] = y
    pltpu.sync_copy(degb, dis_sp.at[pl.ds(row0, NPT)])

    @pl.when(c == 0)
    def _():
        pltpu.sync_copy(degb, dis_hbm.at[pl.ds(row0, NPT)])

    plsc.subcore_barrier()

    vals = (val0, val1)
    sg = (sg0, sg1)
    ss = (ss0, ss1)
    pltpu.async_copy(dis_sp.at[uda.at[0]], val0, sg0)

    def u_pair(j, carry):
        for b in (0, 1):
            kk = 2 * j + b
            nb = 1 - b
            _drain(sg[b], dis_hbm.at[pl.ds(0, CK)], vals[b])
            pltpu.async_copy(vals[b], u_sp.at[usa.at[kk]], ss[b], add=True)

            @pl.when(kk >= 1)
            def _():
                _drain(ss[nb], dis_hbm.at[pl.ds(0, CK)], vals[nb])

            @pl.when(kk + 1 < NCH_HALF)
            def _():
                pltpu.async_copy(dis_sp.at[uda.at[kk + 1]],
                                 vals[nb], sg[nb])
        return carry

    lax.fori_loop(0, NCH_HALF // 2, u_pair, 0)
    _drain(ss[1], dis_hbm.at[pl.ds(0, CK)], val1)
    plsc.subcore_barrier()
    pltpu.sync_copy(u_sp.at[pl.ds(row0, NPT)],
                    up_hbm.at[c, 0, pl.ds(row0, NPT)])


_prep = functools.partial(
    pl.kernel,
    out_type=(jax.ShapeDtypeStruct((NP,), jnp.float32),
              jax.ShapeDtypeStruct((2, 1, NP), jnp.float32)),
    mesh=_sc_mesh(),
    scratch_types=[
        pltpu.VMEM((NPT,), jnp.float32),
        pltpu.VMEM((NPT,), jnp.float32),
        pltpu.VMEM((CK,), jnp.float32),
        pltpu.VMEM((NCH_ALL, CK), jnp.int32),
        pltpu.VMEM((NCH_HALF, CK), jnp.int32),
        pltpu.VMEM((NCH_HALF, CK), jnp.int32),
        pltpu.VMEM((CK,), jnp.float32),
        pltpu.VMEM((CK,), jnp.float32),
        pltpu.SemaphoreType.DMA,
        pltpu.SemaphoreType.DMA,
        pltpu.SemaphoreType.DMA,
        pltpu.SemaphoreType.DMA,
        pltpu.SemaphoreType.DMA,
        pltpu.SemaphoreType.DMA,
        pltpu.VMEM_SHARED((NP,), jnp.float32),
        pltpu.VMEM_SHARED((NP,), jnp.float32),
        pltpu.VMEM_SHARED((NP,), jnp.float32),
    ],
)(_prep_body)


CKS = 128
NCHS = ECH_HALF // CKS
NCHS_H = NCHS // 2


def _row_body(t_hbm, src1_hbm, dst2_hbm, out_hbm,
              srcall, dstall, rows0, rows1,
              semA, semB, sg0, sg1, ss0, ss1, acc_sp):
    c = lax.axis_index("c")
    s = lax.axis_index("s")
    row0 = s * NPT
    wid = c * NTILE + s
    e0 = wid * ECH_HALF
    half_edges = ECH_HALF // 2

    pltpu.async_copy(src1_hbm.at[pl.ds(e0, half_edges)], srcall, semA)
    pltpu.async_copy(dst2_hbm.at[pl.ds(wid * NCHS, NCHS)], dstall, semB)
    pltpu.sync_copy(t_hbm.at[pl.ds(row0, NPT)], acc_sp.at[pl.ds(row0, NPT)])
    _drain(semA, src1_hbm.at[pl.ds(0, half_edges)], srcall)
    _drain(semB, dst2_hbm.at[pl.ds(0, NCHS)], dstall)
    plsc.subcore_barrier()

    rows = (rows0, rows1)
    sg = (sg0, sg1)
    ss = (ss0, ss1)

    for h in (0, 1):
        if h == 1:
            pltpu.async_copy(src1_hbm.at[pl.ds(e0 + half_edges, half_edges)],
                             srcall, semA)
            _drain(semA, src1_hbm.at[pl.ds(0, half_edges)], srcall)
        base = h * NCHS_H
        pltpu.async_copy(t_hbm.at[srcall.at[pl.ds(0, CKS)]], rows0, sg0)

        def pair(j, carry):
            for b in (0, 1):
                lk = 2 * j + b
                nb = 1 - b
                _drain(sg[b], t_hbm.at[pl.ds(0, CKS)], rows[b])
                pltpu.async_copy(rows[b], acc_sp.at[dstall.at[base + lk]],
                                 ss[b], add=True)

                @pl.when(lk >= 1)
                def _():
                    _drain(ss[nb], t_hbm.at[pl.ds(0, CKS)], rows[nb])

                @pl.when(lk + 1 < NCHS_H)
                def _():
                    pltpu.async_copy(
                        t_hbm.at[srcall.at[pl.ds((lk + 1) * CKS, CKS)]],
                        rows[nb], sg[nb])
            return carry

        lax.fori_loop(0, NCHS_H // 2, pair, 0)
        _drain(ss[1], t_hbm.at[pl.ds(0, CKS)], rows1)

    plsc.subcore_barrier()
    pltpu.sync_copy(acc_sp.at[pl.ds(row0, NPT)],
                    out_hbm.at[c, pl.ds(row0, NPT)])


_rowpass = functools.partial(
    pl.kernel,
    out_type=jax.ShapeDtypeStruct((2, NP, D), jnp.float32),
    mesh=_sc_mesh(),
    scratch_types=[
        pltpu.VMEM((ECH_HALF // 2,), jnp.int32),
        pltpu.VMEM((NCHS, CKS), jnp.int32),
        pltpu.VMEM((CKS, D), jnp.float32),
        pltpu.VMEM((CKS, D), jnp.float32),
        pltpu.SemaphoreType.DMA,
        pltpu.SemaphoreType.DMA,
        pltpu.SemaphoreType.DMA,
        pltpu.SemaphoreType.DMA,
        pltpu.SemaphoreType.DMA,
        pltpu.SemaphoreType.DMA,
        pltpu.VMEM_SHARED((NP, D), jnp.float32),
    ],
)(_row_body)


def _mm1_body(x_ref, w_ref, dis_ref, o_ref):
    o_ref[...] = (jnp.dot(x_ref[...], w_ref[...],
                          preferred_element_type=jnp.float32) * dis_ref[...])


def _mm1(x_p, W1, dis_col):
    return pl.pallas_call(
        _mm1_body,
        grid=(GRID,),
        in_specs=[
            pl.BlockSpec((RB, D), lambda i: (i, 0)),
            pl.BlockSpec((D, D), lambda i: (0, 0)),
            pl.BlockSpec((RB, 1), lambda i: (i, 0)),
        ],
        out_specs=pl.BlockSpec((RB, D), lambda i: (i, 0)),
        out_shape=jax.ShapeDtypeStruct((NP, D), jnp.float32),
    )(x_p, W1, dis_col)


def _mid_body(a0_ref, a1_ref, t_ref, dis_ref, u0_ref, u1_ref, b_ref, w_ref,
              t2_ref, v_ref):
    dis = dis_ref[...]
    q = a0_ref[...] + a1_ref[...] - t_ref[...]
    h = jnp.maximum(dis * q + b_ref[...], 0.0)
    t2_ref[...] = jnp.dot(h, w_ref[...],
                          preferred_element_type=jnp.float32) * dis
    v_ref[...] = dis * (u0_ref[...] + u1_ref[...] + dis)


def _mid(acc1, t1, dis_col, u0, u1, b1, W2):
    return pl.pallas_call(
        _mid_body,
        grid=(GRID,),
        in_specs=[
            pl.BlockSpec((RB, D), lambda i: (i, 0)),
            pl.BlockSpec((RB, D), lambda i: (i, 0)),
            pl.BlockSpec((RB, D), lambda i: (i, 0)),
            pl.BlockSpec((RB, 1), lambda i: (i, 0)),
            pl.BlockSpec((RB, 1), lambda i: (i, 0)),
            pl.BlockSpec((RB, 1), lambda i: (i, 0)),
            pl.BlockSpec((1, D), lambda i: (0, 0)),
            pl.BlockSpec((D, D), lambda i: (0, 0)),
        ],
        out_specs=[
            pl.BlockSpec((RB, D), lambda i: (i, 0)),
            pl.BlockSpec((RB, 1), lambda i: (i, 0)),
        ],
        out_shape=[
            jax.ShapeDtypeStruct((NP, D), jnp.float32),
            jax.ShapeDtypeStruct((NP, 1), jnp.float32),
        ],
    )(acc1[0], acc1[1], t1, dis_col, u0, u1, b1, W2)


def _head_body(a0_ref, a1_ref, t_ref, dis_ref, v_ref, b2_ref, w3_ref, b3_ref,
               wc1_ref, bc1_ref, wc2_ref, bc2_ref, acc_ref, o_ref):
    i = pl.program_id(0)
    dis = dis_ref[...]
    q = a0_ref[...] + a1_ref[...] - t_ref[...]
    h = jnp.maximum(dis * q + b2_ref[...], 0.0)
    contrib = jnp.sum(v_ref[...] * h, axis=0, keepdims=True)

    @pl.when(i == 0)
    def _():
        acc_ref[...] = contrib

    @pl.when(i > 0)
    def _():
        acc_ref[...] = acc_ref[...] + contrib

    @pl.when(i == GRID - 1)
    def _():
        g = (jnp.dot(acc_ref[...], w3_ref[...],
                     preferred_element_type=jnp.float32) * (1.0 / N)
             + b3_ref[...])
        z = jnp.maximum(jnp.dot(g, wc1_ref[...],
                                preferred_element_type=jnp.float32)
                        + bc1_ref[...], 0.0)
        o = jnp.dot(z, wc2_ref[...],
                    preferred_element_type=jnp.float32) + bc2_ref[...]
        o_ref[...] = 1.0 / (1.0 + jnp.exp(-o))


def _head(acc2, t2, dis_col, v_col, b2, W3, b3, Wc1, bc1, Wc2p, bc2p):
    return pl.pallas_call(
        _head_body,
        grid=(GRID,),
        in_specs=[
            pl.BlockSpec((RB, D), lambda i: (i, 0)),
            pl.BlockSpec((RB, D), lambda i: (i, 0)),
            pl.BlockSpec((RB, D), lambda i: (i, 0)),
            pl.BlockSpec((RB, 1), lambda i: (i, 0)),
            pl.BlockSpec((RB, 1), lambda i: (i, 0)),
            pl.BlockSpec((1, D), lambda i: (0, 0)),
            pl.BlockSpec((D, D), lambda i: (0, 0)),
            pl.BlockSpec((1, D), lambda i: (0, 0)),
            pl.BlockSpec((D, 64), lambda i: (0, 0)),
            pl.BlockSpec((1, 64), lambda i: (0, 0)),
            pl.BlockSpec((64, D), lambda i: (0, 0)),
            pl.BlockSpec((1, D), lambda i: (0, 0)),
        ],
        out_specs=[
            pl.BlockSpec((1, D), lambda i: (0, 0)),
            pl.BlockSpec((1, D), lambda i: (0, 0)),
        ],
        out_shape=[
            jax.ShapeDtypeStruct((1, D), jnp.float32),
            jax.ShapeDtypeStruct((1, D), jnp.float32),
        ],
    )(acc2[0], acc2[1], t2, dis_col, v_col, b2, W3, b3, Wc1, bc1, Wc2p, bc2p)


def kernel(x, edge_index, W1, b1, W2, b2, W3, b3, Wc1, bc1, Wc2, bc2):
    src = edge_index[0]
    dst = edge_index[1]
    padv = (N + jnp.arange(EP - E, dtype=jnp.int32) % (NP - N))
    src_p = jnp.concatenate([src, padv])
    dst_p = jnp.concatenate([dst, padv])
    src2 = src_p.reshape(EP // CK, CK)
    dst2 = dst_p.reshape(EP // CK, CK)
    dst2b = dst2
    x_p = jnp.pad(x, ((0, NP - N), (0, 0)))

    dis, up = _prep(dst2, src2)
    dis_col = dis.reshape(NP, 1)
    u0 = up[0, 0].reshape(NP, 1)
    u1 = up[1, 0].reshape(NP, 1)

    t1 = _mm1(x_p, W1, dis_col)
    acc1 = _rowpass(t1, src_p, dst2b)
    t2, v_col = _mid(acc1, t1, dis_col, u0, u1, b1.reshape(1, D), W2)
    acc2 = _rowpass(t2, src_p, dst2b)

    Wc2p = jnp.pad(Wc2, ((0, 0), (0, D - 6)))
    bc2p = jnp.pad(bc2, (0, D - 6)).reshape(1, D)
    _, outp = _head(acc2, t2, dis_col, v_col, b2.reshape(1, D), W3,
                    b3.reshape(1, D), Wc1, bc1.reshape(1, 64), Wc2p, bc2p)
    return outp[:, :6]
